# baseline (speedup 1.0000x reference)
import functools

import jax
import jax.numpy as jnp
from jax.experimental import pallas as pl
from jax.experimental.pallas import tpu as pltpu

B, S, H = 1, 2048, 768
E, K, INTER = 8, 2, 768
ALPHA, LIMIT = 1.702, 7.0

T = B * S
TT2 = 256
G2 = T * K // TT2 + E
P = G2 * TT2


def _router_kernel(x_ref, rw_ref, rb_ref, scores_ref, idx_ref, w_ref):
    x = x_ref[...]
    logits = jax.lax.dot_general(
        x.astype(jnp.bfloat16), rw_ref[...].astype(jnp.bfloat16),
        (((1,), (1,)), ((), ())),
        preferred_element_type=jnp.float32)
    logits = logits + rb_ref[...]
    n = logits.shape[0]
    idx = jax.lax.broadcasted_iota(jnp.int32, (n, E), 1)
    m1 = jnp.max(logits, axis=1, keepdims=True)
    i1 = jnp.min(jnp.where(logits == m1, idx, E), axis=1, keepdims=True)
    sel1 = idx == i1
    masked = jnp.where(sel1, -jnp.inf, logits)
    m2 = jnp.max(masked, axis=1, keepdims=True)
    i2 = jnp.min(jnp.where(masked == m2, idx, E), axis=1, keepdims=True)
    sel2 = idx == i2
    e2 = jnp.exp(m2 - m1)
    denom = 1.0 + e2
    p1 = 1.0 / denom
    p2 = e2 / denom
    scores_ref[...] = jnp.where(sel1, p1, 0.0) + jnp.where(sel2, p2, 0.0)
    idx_ref[...] = jnp.concatenate([i1, i2], axis=1)
    w_ref[...] = jnp.concatenate([p1, p2], axis=1)


def _gmm_kernel(te_ref, xs_ref, gup_ref, gub_ref, dp_ref, db_ref, w_ref,
                dd_ref, gupb_s, dp2_s):
    i = pl.program_id(0)
    te = te_ref[i]
    te_prev = te_ref[jnp.maximum(i - 1, 0)]
    fresh = jnp.logical_or(i == 0, te != te_prev)

    @pl.when(jnp.logical_and(te >= 0, fresh))
    def _load_weights():
        gupb_s[...] = gup_ref[0].astype(jnp.bfloat16)
        dpv = dp_ref[0].astype(jnp.bfloat16)
        dp2_s[...] = jnp.concatenate(
            [dpv[:, None, :], jnp.zeros_like(dpv)[:, None, :]],
            axis=1).reshape(2 * INTER, H)

    @pl.when(te >= 0)
    def _compute():
        xb = xs_ref[...].astype(jnp.bfloat16)
        gu = jnp.dot(xb, gupb_s[...],
                     preferred_element_type=jnp.float32) + gub_ref[0]
        gu_up = pltpu.roll(gu, 2 * INTER - 1, 1)
        gate = jnp.minimum(gu, LIMIT)
        up = jnp.clip(gu_up, -LIMIT, LIMIT)
        glu = gate * jax.nn.sigmoid(gate * ALPHA)
        act = (up + 1.0) * glu
        lane = jax.lax.broadcasted_iota(jnp.int32, (TT2, 2 * INTER), 1)
        act = jnp.where(lane % 2 == 0, act, 0.0)
        mm = jnp.dot(act.astype(jnp.bfloat16), dp2_s[...],
                     preferred_element_type=jnp.float32) + db_ref[0]
        dd_ref[...] = mm * w_ref[...]

    @pl.when(te < 0)
    def _pad():
        dd_ref[...] = jnp.zeros((TT2, H), jnp.float32)


@functools.partial(jax.jit, static_argnames=())
def kernel(hidden_states, router_weight, router_bias, gate_up_proj,
           gate_up_proj_bias, down_proj, down_proj_bias):
    b, s, h = hidden_states.shape
    x = hidden_states.reshape(T, h)
    rb = router_bias.reshape(1, E)
    gub = gate_up_proj_bias.reshape(E, 1, 2 * INTER)
    db = down_proj_bias.reshape(E, 1, H)

    scores, topk_idx, topk_w = pl.pallas_call(
        _router_kernel,
        grid=(1,),
        in_specs=[
            pl.BlockSpec((T, H), lambda i: (0, 0)),
            pl.BlockSpec((E, H), lambda i: (0, 0)),
            pl.BlockSpec((1, E), lambda i: (0, 0)),
        ],
        out_specs=[
            pl.BlockSpec((T, E), lambda i: (0, 0)),
            pl.BlockSpec((T, K), lambda i: (0, 0)),
            pl.BlockSpec((T, K), lambda i: (0, 0)),
        ],
        out_shape=[
            jax.ShapeDtypeStruct((T, E), jnp.float32),
            jax.ShapeDtypeStruct((T, K), jnp.int32),
            jax.ShapeDtypeStruct((T, K), jnp.float32),
        ],
    )(x, router_weight, rb)

    flat_e = topk_idx.reshape(-1)
    flat_w = topk_w.reshape(-1)
    n = T * K
    oh = (flat_e[:, None] == jnp.arange(E)[None, :]).astype(jnp.int32)
    ranks = jnp.cumsum(oh, axis=0) - oh
    counts = jnp.sum(oh, axis=0)
    gt = (counts + TT2 - 1) // TT2
    pstart = jnp.concatenate(
        [jnp.zeros(1, jnp.int32),
         jnp.cumsum(gt * TT2)])[:E].astype(jnp.int32)
    rank = jnp.take_along_axis(ranks, flat_e[:, None], axis=1)[:, 0]
    dest = pstart[flat_e] + rank
    tok = (jnp.arange(n, dtype=jnp.int32) // K)
    row_token = jnp.zeros(P, jnp.int32).at[dest].set(tok)
    row_w = jnp.zeros(P, jnp.float32).at[dest].set(flat_w)
    pos = dest.reshape(T, K)
    tile_starts = pstart // TT2
    ti = jnp.arange(G2, dtype=jnp.int32)
    in_e = (ti[:, None] >= tile_starts[None, :]) & (
        ti[:, None] < (tile_starts + gt)[None, :])
    tile_expert = jnp.where(
        in_e.any(axis=1), jnp.argmax(in_e, axis=1), -1).astype(jnp.int32)

    x_sorted = jnp.take(x, row_token, axis=0)

    dd = pl.pallas_call(
        _gmm_kernel,
        grid_spec=pltpu.PrefetchScalarGridSpec(
            num_scalar_prefetch=1,
            grid=(G2,),
            in_specs=[
                pl.BlockSpec((TT2, H), lambda i, te: (i, 0)),
                pl.BlockSpec((1, H, 2 * INTER),
                             lambda i, te: (jnp.maximum(te[i], 0), 0, 0)),
                pl.BlockSpec((1, 1, 2 * INTER),
                             lambda i, te: (jnp.maximum(te[i], 0), 0, 0)),
                pl.BlockSpec((1, INTER, H),
                             lambda i, te: (jnp.maximum(te[i], 0), 0, 0)),
                pl.BlockSpec((1, 1, H),
                             lambda i, te: (jnp.maximum(te[i], 0), 0, 0)),
                pl.BlockSpec((TT2, 1), lambda i, te: (i, 0)),
            ],
            out_specs=pl.BlockSpec((TT2, H), lambda i, te: (i, 0)),
            scratch_shapes=[
                pltpu.VMEM((H, 2 * INTER), jnp.bfloat16),
                pltpu.VMEM((2 * INTER, H), jnp.bfloat16),
            ],
        ),
        out_shape=jax.ShapeDtypeStruct((P, H), jnp.float32),
        compiler_params=pltpu.CompilerParams(
            dimension_semantics=("arbitrary",),
        ),
    )(tile_expert, x_sorted, gate_up_proj, gub, down_proj, db,
      row_w.reshape(P, 1))

    out = jnp.take(dd, pos[:, 0], axis=0) + jnp.take(dd, pos[:, 1], axis=0)
    return out.reshape(b, s, h), scores


# speedup vs baseline: 1.4356x; 1.4356x over previous
import functools

import jax
import jax.numpy as jnp
from jax.experimental import pallas as pl
from jax.experimental.pallas import tpu as pltpu

B, S, H = 1, 2048, 768
E, K, INTER = 8, 2, 768
ALPHA, LIMIT = 1.702, 7.0

T = B * S
JB = INTER


def _moe_kernel(x_ref, rw_ref, rb_ref, gup_ref, gub_ref, dp_ref, db_ref,
                out_ref, scores_ref):
    e = pl.program_id(0)
    j = pl.program_id(1)
    first = jnp.logical_and(e == 0, j == 0)
    x = x_ref[...]

    @pl.when(first)
    def _router():
        logits = jax.lax.dot_general(
            x.astype(jnp.bfloat16), rw_ref[...].astype(jnp.bfloat16),
            (((1,), (1,)), ((), ())),
            preferred_element_type=jnp.float32)
        logits = logits + rb_ref[...]
        idx = jax.lax.broadcasted_iota(jnp.int32, (T, E), 1)
        m1 = jnp.max(logits, axis=1, keepdims=True)
        i1 = jnp.min(jnp.where(logits == m1, idx, E), axis=1, keepdims=True)
        sel1 = idx == i1
        masked = jnp.where(sel1, -jnp.inf, logits)
        m2 = jnp.max(masked, axis=1, keepdims=True)
        i2 = jnp.min(jnp.where(masked == m2, idx, E), axis=1, keepdims=True)
        sel2 = idx == i2
        e2 = jnp.exp(m2 - m1)
        denom = 1.0 + e2
        p1 = 1.0 / denom
        p2 = e2 / denom
        scores_ref[...] = jnp.where(sel1, p1, 0.0) + jnp.where(sel2, p2, 0.0)

    scores = scores_ref[...]
    idx = jax.lax.broadcasted_iota(jnp.int32, (T, E), 1)
    w_e = jnp.sum(jnp.where(idx == e, scores, 0.0), axis=1, keepdims=True)

    xb = x.astype(jnp.bfloat16)
    gu = jnp.dot(xb, gup_ref[0].astype(jnp.bfloat16),
                 preferred_element_type=jnp.float32) + gub_ref[0]
    gu_up = pltpu.roll(gu, JB - 1, 1)
    gate = jnp.minimum(gu, LIMIT)
    up = jnp.clip(gu_up, -LIMIT, LIMIT)
    glu = gate * jax.nn.sigmoid(gate * ALPHA)
    act = (up + 1.0) * glu
    lane = jax.lax.broadcasted_iota(jnp.int32, (T, JB), 1)
    act = jnp.where(lane % 2 == 0, act, 0.0)
    dpv = dp_ref[0].astype(jnp.bfloat16)
    dp2 = jnp.concatenate(
        [dpv[:, None, :], jnp.zeros_like(dpv)[:, None, :]],
        axis=1).reshape(JB, H)
    mm = jnp.dot(act.astype(jnp.bfloat16), dp2,
                 preferred_element_type=jnp.float32)
    contrib = jnp.where(j == 0, mm + db_ref[0], mm) * w_e

    @pl.when(first)
    def _init():
        out_ref[...] = contrib

    @pl.when(jnp.logical_not(first))
    def _acc():
        out_ref[...] += contrib


@functools.partial(jax.jit, static_argnames=())
def kernel(hidden_states, router_weight, router_bias, gate_up_proj,
           gate_up_proj_bias, down_proj, down_proj_bias):
    b, s, h = hidden_states.shape
    x = hidden_states.reshape(T, h)
    gub = gate_up_proj_bias.reshape(E, 1, 2 * INTER)
    db = down_proj_bias.reshape(E, 1, H)
    rb = router_bias.reshape(1, E)

    grid = (E, 2)
    out, scores = pl.pallas_call(
        _moe_kernel,
        grid=grid,
        in_specs=[
            pl.BlockSpec((T, H), lambda e, j: (0, 0)),
            pl.BlockSpec((E, H), lambda e, j: (0, 0)),
            pl.BlockSpec((1, E), lambda e, j: (0, 0)),
            pl.BlockSpec((1, H, JB), lambda e, j: (e, 0, j)),
            pl.BlockSpec((1, 1, JB), lambda e, j: (e, 0, j)),
            pl.BlockSpec((1, JB // 2, H), lambda e, j: (e, j, 0)),
            pl.BlockSpec((1, 1, H), lambda e, j: (e, 0, 0)),
        ],
        out_specs=[
            pl.BlockSpec((T, H), lambda e, j: (0, 0)),
            pl.BlockSpec((T, E), lambda e, j: (0, 0)),
        ],
        out_shape=[
            jax.ShapeDtypeStruct((T, H), jnp.float32),
            jax.ShapeDtypeStruct((T, E), jnp.float32),
        ],
        compiler_params=pltpu.CompilerParams(
            dimension_semantics=("arbitrary", "arbitrary"),
        ),
    )(x, router_weight, rb, gate_up_proj, gub, down_proj, db)

    return out.reshape(b, s, h), scores


# speedup vs baseline: 1.4622x; 1.0185x over previous
import functools

import jax
import jax.numpy as jnp
from jax.experimental import pallas as pl
from jax.experimental.pallas import tpu as pltpu

B, S, H = 1, 2048, 768
E, K, INTER = 8, 2, 768
ALPHA, LIMIT = 1.702, 7.0

TT = 1024


def _moe_kernel(x_ref, rw_ref, rb_ref, gup_ref, gub_ref, dp_ref, db_ref,
                out_ref, scores_ref):
    e = pl.program_id(1)
    x = x_ref[...]

    @pl.when(e == 0)
    def _router():
        logits = jax.lax.dot_general(
            x.astype(jnp.bfloat16), rw_ref[...].astype(jnp.bfloat16),
            (((1,), (1,)), ((), ())),
            preferred_element_type=jnp.float32)
        logits = logits + rb_ref[...]
        idx = jax.lax.broadcasted_iota(jnp.int32, (TT, E), 1)
        m1 = jnp.max(logits, axis=1, keepdims=True)
        i1 = jnp.min(jnp.where(logits == m1, idx, E), axis=1, keepdims=True)
        sel1 = idx == i1
        masked = jnp.where(sel1, -jnp.inf, logits)
        m2 = jnp.max(masked, axis=1, keepdims=True)
        i2 = jnp.min(jnp.where(masked == m2, idx, E), axis=1, keepdims=True)
        sel2 = idx == i2
        e2 = jnp.exp(m2 - m1)
        denom = 1.0 + e2
        p1 = 1.0 / denom
        p2 = e2 / denom
        scores_ref[...] = jnp.where(sel1, p1, 0.0) + jnp.where(sel2, p2, 0.0)

    scores = scores_ref[...]
    idx = jax.lax.broadcasted_iota(jnp.int32, (TT, E), 1)
    w_e = jnp.sum(jnp.where(idx == e, scores, 0.0), axis=1, keepdims=True)

    xb = x.astype(jnp.bfloat16)
    gu = jnp.dot(xb, gup_ref[0].astype(jnp.bfloat16),
                 preferred_element_type=jnp.float32) + gub_ref[0]
    gu_up = pltpu.roll(gu, 2 * INTER - 1, 1)
    gate = jnp.minimum(gu, LIMIT)
    up = jnp.clip(gu_up, -LIMIT, LIMIT)
    glu = gate * jax.nn.sigmoid(gate * ALPHA)
    act = (up + 1.0) * glu
    lane = jax.lax.broadcasted_iota(jnp.int32, (TT, 2 * INTER), 1)
    act = jnp.where(lane % 2 == 0, act, 0.0)
    dpv = dp_ref[0].astype(jnp.bfloat16)
    dp2 = jnp.concatenate(
        [dpv[:, None, :], jnp.zeros_like(dpv)[:, None, :]],
        axis=1).reshape(2 * INTER, H)
    dd = jnp.dot(act.astype(jnp.bfloat16), dp2,
                 preferred_element_type=jnp.float32)
    dd = dd + db_ref[0]
    contrib = dd * w_e

    @pl.when(e == 0)
    def _init():
        out_ref[...] = contrib

    @pl.when(e != 0)
    def _acc():
        out_ref[...] += contrib


@functools.partial(jax.jit, static_argnames=())
def kernel(hidden_states, router_weight, router_bias, gate_up_proj,
           gate_up_proj_bias, down_proj, down_proj_bias):
    b, s, h = hidden_states.shape
    T = b * s
    x = hidden_states.reshape(T, h)
    gup = gate_up_proj
    gub = gate_up_proj_bias.reshape(E, 1, 2 * INTER)
    dp = down_proj
    db = down_proj_bias.reshape(E, 1, H)
    rb = router_bias.reshape(1, E)

    n_t = T // TT
    grid = (n_t, E)
    out, scores = pl.pallas_call(
        _moe_kernel,
        grid=grid,
        in_specs=[
            pl.BlockSpec((TT, H), lambda t, e: (t, 0)),
            pl.BlockSpec((E, H), lambda t, e: (0, 0)),
            pl.BlockSpec((1, E), lambda t, e: (0, 0)),
            pl.BlockSpec((1, H, 2 * INTER), lambda t, e: (e, 0, 0)),
            pl.BlockSpec((1, 1, 2 * INTER), lambda t, e: (e, 0, 0)),
            pl.BlockSpec((1, INTER, H), lambda t, e: (e, 0, 0)),
            pl.BlockSpec((1, 1, H), lambda t, e: (e, 0, 0)),
        ],
        out_specs=[
            pl.BlockSpec((TT, H), lambda t, e: (t, 0)),
            pl.BlockSpec((TT, E), lambda t, e: (t, 0)),
        ],
        out_shape=[
            jax.ShapeDtypeStruct((T, H), jnp.float32),
            jax.ShapeDtypeStruct((T, E), jnp.float32),
        ],
        compiler_params=pltpu.CompilerParams(
            dimension_semantics=("parallel", "arbitrary"),
        ),
    )(x, router_weight, rb, gup, gub, dp, db)

    return out.reshape(b, s, h), scores
